# baseline (speedup 1.0000x reference)
import functools

import jax
import jax.numpy as jnp
from jax import lax
from jax.experimental import pallas as pl
from jax.experimental.pallas import tpu as pltpu
from jax.experimental.pallas import tpu_sc as plsc

B, C, H, W, Q = 4, 192, 96, 96, 8192
HW = H * W
NCORE, NSUB = 2, 16
CHG = C // NSUB
NPAIR = CHG // 2
BPG = B // NCORE
QC = 256
NG = QC // 16
NCHUNK = Q // QC
PACK_UNROLL = 8
_IL = plsc.PackFormat.INTERLEAVED


@functools.lru_cache(maxsize=1)
def _build():
    mesh = plsc.VectorSubcoreMesh(
        core_axis_name="c", subcore_axis_name="s",
        num_cores=NCORE, num_subcores=NSUB)
    return functools.partial(
        pl.kernel,
        out_type=jax.ShapeDtypeStruct((B, C, 4, Q), jnp.float32),
        mesh=mesh,
        compiler_params=pltpu.CompilerParams(needs_layout_passes=False),
        scratch_types=[
            pltpu.VMEM((NPAIR * HW,), jnp.int32),
            pltpu.VMEM((2 * HW,), jnp.float32),
            pltpu.VMEM((CHG, 4, QC), jnp.float32),
            pltpu.VMEM((CHG, 4, QC), jnp.float32),
            pltpu.VMEM((Q * 3,), jnp.float32),
            pltpu.VMEM((16,), jnp.float32),
            pltpu.VMEM((16,), jnp.float32),
            pltpu.SemaphoreType.DMA,
            pltpu.SemaphoreType.DMA,
        ],
    )(_dgs_sc)


def _dgs_sc(feat_hbm, grid_hbm, fsw_hbm, fsh_hbm, out_hbm,
            feat_v, fraw_v, stage_a, stage_b, grid_v, fswv, fshv,
            sem_a, sem_b):
    cid = lax.axis_index("c")
    sid = lax.axis_index("s")
    cg = sid
    bp = cid
    lane = lax.broadcasted_iota(jnp.int32, (16,), 0)
    lane3 = lane * 3

    def batch_body(bi, _):
        b = bp * BPG + bi
        pltpu.sync_copy(fsw_hbm.at[b], fswv)
        pltpu.sync_copy(fsh_hbm.at[b], fshv)
        fw = fswv[...]
        fh = fshv[...]

        for p in range(NPAIR):
            pltpu.sync_copy(
                feat_hbm.at[pl.ds((b * C + cg * CHG + 2 * p) * HW, 2 * HW)],
                fraw_v)

            def pack_body(i, _, p=p):
                o = i * (16 * PACK_UNROLL)
                for u in range(PACK_UNROLL):
                    oo = o + u * 16
                    a = fraw_v[pl.ds(oo, 16)]
                    bb = fraw_v[pl.ds(HW + oo, 16)]
                    feat_v[pl.ds(p * HW + oo, 16)] = plsc.bitcast(
                        plsc.pack(a, bb, format=_IL), jnp.int32)
                return 0

            lax.fori_loop(0, HW // (16 * PACK_UNROLL), pack_body, 0)

        pltpu.sync_copy(grid_hbm.at[pl.ds(b * Q * 3, Q * 3)], grid_v)

        def fill(ch, stage):
            q0 = ch * QC

            @plsc.parallel_loop(0, NG // 2)
            def group_body(g):
              for sub in range(2):
                off = g * 32 + sub * 16
                gidx = lane3 + (q0 + off) * 3
                xq = plsc.load_gather(grid_v, [gidx])
                yq = plsc.load_gather(grid_v, [gidx + 1])
                zq = plsc.load_gather(grid_v, [gidx + 2])
                ix = jnp.clip((xq + 1.0) * (0.5 * (W - 1)), 0.0, W - 1.0)
                iy = jnp.clip((yq + 1.0) * (0.5 * (H - 1)), 0.0, H - 1.0)
                x0i = jnp.minimum(ix.astype(jnp.int32), W - 2)
                y0i = jnp.minimum(iy.astype(jnp.int32), H - 2)
                dx = ix - x0i.astype(jnp.float32)
                dy = iy - y0i.astype(jnp.float32)
                pix = y0i * W + x0i
                rz = 1.0 / zq
                sw = fw * rz
                sh = fh * rz
                nsx = -(xq * rz)
                nsy = -(yq * rz)
                dxp = plsc.pack(dx, dx, format=_IL)
                dyp = plsc.pack(dy, dy, format=_IL)
                swp = plsc.pack(sw, sw, format=_IL)
                shp = plsc.pack(sh, sh, format=_IL)
                nsxp = plsc.pack(nsx, nsx, format=_IL)
                nsyp = plsc.pack(nsy, nsy, format=_IL)
                def fetch(p):
                    base = pix + (p * HW)
                    return (plsc.load_gather(feat_v, [base]),
                            plsc.load_gather(feat_v, [base + 1]),
                            plsc.load_gather(feat_v, [base + W]),
                            plsc.load_gather(feat_v, [base + (W + 1)]))

                cur = fetch(0)
                for p in range(NPAIR):
                    nxt = fetch(p + 1) if p + 1 < NPAIR else None
                    f00, f01, f10, f11 = (
                        plsc.bitcast(w, jnp.bfloat16) for w in cur)
                    g0 = f01 - f00
                    g1 = f11 - f10
                    t0 = f00 + dxp * g0
                    t1 = f10 + dxp * g1
                    phi = t0 + dyp * (t1 - t0)
                    dj = g0 + dyp * (g1 - g0)
                    h0 = f10 - f00
                    h1 = f11 - f01
                    di = h0 + dxp * (h1 - h0)
                    o1 = dj * swp
                    o2 = di * shp
                    o3 = di * nsyp + dj * nsxp
                    for d, val in enumerate((phi, o1, o2, o3)):
                        ea, ob = plsc.unpack(val, format=_IL)
                        stage[2 * p, d, pl.ds(off, 16)] = ea
                        stage[2 * p + 1, d, pl.ds(off, 16)] = ob
                    cur = nxt

        def out_slice(ch):
            return out_hbm.at[b, pl.ds(cg * CHG, CHG), :, pl.ds(ch * QC, QC)]

        fill(0, stage_a)
        pltpu.async_copy(stage_a, out_slice(0), sem_a)
        fill(1, stage_b)
        pltpu.async_copy(stage_b, out_slice(1), sem_b)

        def chunk_pair_body(j, _):
            ch = 2 * j + 2
            pltpu.make_async_copy(stage_a, out_slice(0), sem_a).wait()
            fill(ch, stage_a)
            pltpu.async_copy(stage_a, out_slice(ch), sem_a)
            pltpu.make_async_copy(stage_b, out_slice(1), sem_b).wait()
            fill(ch + 1, stage_b)
            pltpu.async_copy(stage_b, out_slice(ch + 1), sem_b)
            return 0

        lax.fori_loop(0, (NCHUNK - 2) // 2, chunk_pair_body, 0)
        pltpu.make_async_copy(stage_a, out_slice(0), sem_a).wait()
        pltpu.make_async_copy(stage_b, out_slice(1), sem_b).wait()
        return 0

    lax.fori_loop(0, BPG, batch_body, 0)


def kernel(input, grid, fScaleWidth, fScaleHeight):
    feat = input.reshape(B * C * HW)
    gridf = grid.reshape(B * Q * 3)
    fsw = jnp.broadcast_to(fScaleWidth[:, None], (B, 16))
    fsh = jnp.broadcast_to(fScaleHeight[:, None], (B, 16))
    return _build()(feat, gridf, fsw, fsh)


# speedup vs baseline: 1.1409x; 1.1409x over previous
import functools

import jax
import jax.numpy as jnp
from jax import lax
from jax.experimental import pallas as pl
from jax.experimental.pallas import tpu as pltpu
from jax.experimental.pallas import tpu_sc as plsc

B, C, H, W, Q = 4, 192, 96, 96, 8192
HW = H * W
NCORE, NSUB = 2, 16
CHG = C // NSUB
NPAIR = CHG // 2
BPG = B // NCORE
QC = 128
NG = QC // 16
NCHUNK = Q // QC
PACK_UNROLL = 8
_IL = plsc.PackFormat.INTERLEAVED


@functools.lru_cache(maxsize=1)
def _build():
    mesh = plsc.VectorSubcoreMesh(
        core_axis_name="c", subcore_axis_name="s",
        num_cores=NCORE, num_subcores=NSUB)
    return functools.partial(
        pl.kernel,
        out_type=jax.ShapeDtypeStruct((B, C, 4, Q), jnp.float32),
        mesh=mesh,
        compiler_params=pltpu.CompilerParams(needs_layout_passes=False),
        scratch_types=[
            pltpu.VMEM((NPAIR * HW,), jnp.int32),
            pltpu.VMEM((2 * HW,), jnp.float32),
            pltpu.VMEM((2 * HW,), jnp.float32),
            pltpu.VMEM((CHG, 4, QC), jnp.float32),
            pltpu.VMEM((CHG, 4, QC), jnp.float32),
            pltpu.VMEM((Q * 3,), jnp.float32),
            pltpu.VMEM((16,), jnp.float32),
            pltpu.VMEM((16,), jnp.float32),
            pltpu.SemaphoreType.DMA,
            pltpu.SemaphoreType.DMA,
            pltpu.SemaphoreType.DMA,
            pltpu.SemaphoreType.DMA,
            pltpu.SemaphoreType.DMA,
        ],
    )(_dgs_sc)


def _dgs_sc(feat_hbm, grid_hbm, fsw_hbm, fsh_hbm, out_hbm,
            feat_v, fraw_a, fraw_b, stage_a, stage_b, grid_v, fswv, fshv,
            sem_a, sem_b, sem_fa, sem_fb, sem_g):
    cid = lax.axis_index("c")
    sid = lax.axis_index("s")
    cg = sid
    bp = cid
    lane = lax.broadcasted_iota(jnp.int32, (16,), 0)
    lane3 = lane * 3

    def batch_body(bi, _):
        b = bp * BPG + bi
        pltpu.sync_copy(fsw_hbm.at[b], fswv)
        pltpu.sync_copy(fsh_hbm.at[b], fshv)
        fw = fswv[...]
        fh = fshv[...]

        pltpu.async_copy(grid_hbm.at[pl.ds(b * Q * 3, Q * 3)], grid_v, sem_g)

        def pair_slice(p):
            return feat_hbm.at[pl.ds((b * C + cg * CHG + 2 * p) * HW, 2 * HW)]

        rawbufs = (fraw_a, fraw_b)
        rawsems = (sem_fa, sem_fb)
        pltpu.async_copy(pair_slice(0), fraw_a, sem_fa)
        for p in range(NPAIR):
            buf = rawbufs[p % 2]
            pltpu.make_async_copy(pair_slice(p), buf, rawsems[p % 2]).wait()
            if p + 1 < NPAIR:
                pltpu.async_copy(pair_slice(p + 1), rawbufs[(p + 1) % 2],
                                 rawsems[(p + 1) % 2])

            def pack_body(i, _, p=p, buf=buf):
                o = i * (16 * PACK_UNROLL)
                for u in range(PACK_UNROLL):
                    oo = o + u * 16
                    a = buf[pl.ds(oo, 16)]
                    bb = buf[pl.ds(HW + oo, 16)]
                    feat_v[pl.ds(p * HW + oo, 16)] = plsc.bitcast(
                        plsc.pack(a, bb, format=_IL), jnp.int32)
                return 0

            lax.fori_loop(0, HW // (16 * PACK_UNROLL), pack_body, 0)

        pltpu.make_async_copy(grid_hbm.at[pl.ds(0, Q * 3)], grid_v,
                              sem_g).wait()

        def fill(ch, stage):
            q0 = ch * QC

            def group_body(g, _):
              for sub in range(2):
                off = g * 32 + sub * 16
                gidx = lane3 + (q0 + off) * 3
                xq = plsc.load_gather(grid_v, [gidx])
                yq = plsc.load_gather(grid_v, [gidx + 1])
                zq = plsc.load_gather(grid_v, [gidx + 2])
                ix = jnp.clip((xq + 1.0) * (0.5 * (W - 1)), 0.0, W - 1.0)
                iy = jnp.clip((yq + 1.0) * (0.5 * (H - 1)), 0.0, H - 1.0)
                x0i = jnp.minimum(ix.astype(jnp.int32), W - 2)
                y0i = jnp.minimum(iy.astype(jnp.int32), H - 2)
                dx = ix - x0i.astype(jnp.float32)
                dy = iy - y0i.astype(jnp.float32)
                pix = y0i * W + x0i
                rz = 1.0 / zq
                sw = fw * rz
                sh = fh * rz
                nsx = -(xq * rz)
                nsy = -(yq * rz)
                dxp = plsc.pack(dx, dx, format=_IL)
                dyp = plsc.pack(dy, dy, format=_IL)
                swp = plsc.pack(sw, sw, format=_IL)
                shp = plsc.pack(sh, sh, format=_IL)
                nsxp = plsc.pack(nsx, nsx, format=_IL)
                nsyp = plsc.pack(nsy, nsy, format=_IL)
                def fetch(p):
                    base = pix + (p * HW)
                    return (plsc.load_gather(feat_v, [base]),
                            plsc.load_gather(feat_v, [base + 1]),
                            plsc.load_gather(feat_v, [base + W]),
                            plsc.load_gather(feat_v, [base + (W + 1)]))

                cur = fetch(0)
                for p in range(NPAIR):
                    nxt = fetch(p + 1) if p + 1 < NPAIR else None
                    f00, f01, f10, f11 = (
                        plsc.bitcast(w, jnp.bfloat16) for w in cur)
                    g0 = f01 - f00
                    g1 = f11 - f10
                    t0 = f00 + dxp * g0
                    t1 = f10 + dxp * g1
                    phi = t0 + dyp * (t1 - t0)
                    dj = g0 + dyp * (g1 - g0)
                    h0 = f10 - f00
                    h1 = f11 - f01
                    di = h0 + dxp * (h1 - h0)
                    o1 = dj * swp
                    o2 = di * shp
                    o3 = di * nsyp + dj * nsxp
                    for d, val in enumerate((phi, o1, o2, o3)):
                        ea, ob = plsc.unpack(val, format=_IL)
                        stage[2 * p, d, pl.ds(off, 16)] = ea
                        stage[2 * p + 1, d, pl.ds(off, 16)] = ob
                    cur = nxt
              return 0

            lax.fori_loop(0, NG // 2, group_body, 0)

        def out_slice(ch):
            return out_hbm.at[b, pl.ds(cg * CHG, CHG), :, pl.ds(ch * QC, QC)]

        fill(0, stage_a)
        pltpu.async_copy(stage_a, out_slice(0), sem_a)
        fill(1, stage_b)
        pltpu.async_copy(stage_b, out_slice(1), sem_b)

        def chunk_pair_body(j, _):
            ch = 2 * j + 2
            pltpu.make_async_copy(stage_a, out_slice(0), sem_a).wait()
            fill(ch, stage_a)
            pltpu.async_copy(stage_a, out_slice(ch), sem_a)
            pltpu.make_async_copy(stage_b, out_slice(1), sem_b).wait()
            fill(ch + 1, stage_b)
            pltpu.async_copy(stage_b, out_slice(ch + 1), sem_b)
            return 0

        lax.fori_loop(0, (NCHUNK - 2) // 2, chunk_pair_body, 0)
        pltpu.make_async_copy(stage_a, out_slice(0), sem_a).wait()
        pltpu.make_async_copy(stage_b, out_slice(1), sem_b).wait()
        return 0

    lax.fori_loop(0, BPG, batch_body, 0)


def kernel(input, grid, fScaleWidth, fScaleHeight):
    feat = input.reshape(B * C * HW)
    gridf = grid.reshape(B * Q * 3)
    fsw = jnp.broadcast_to(fScaleWidth[:, None], (B, 16))
    fsh = jnp.broadcast_to(fScaleHeight[:, None], (B, 16))
    return _build()(feat, gridf, fsw, fsh)
